# Initial kernel scaffold; baseline (speedup 1.0000x reference)
import jax
import jax.numpy as jnp
from jax.experimental import pallas as pl


def kernel(x, table):
    raise NotImplementedError("write your pallas kernel here")


# speedup vs baseline: 1.1298x; 1.1298x over previous
import functools

import jax
import jax.numpy as jnp
from jax import lax
from jax.experimental import pallas as pl
from jax.experimental.pallas import tpu as pltpu
from jax.experimental.pallas import tpu_sc as plsc

HIDDEN_DIM = 32
MAX_NORM = 1.0

B_ROWS = 16384 * 26
NC, NS, LANES = 2, 16, 16
NW = NC * NS
ROWS_PER_W = B_ROWS // NW
CHUNK = 1024
GROUPS = CHUNK // 128
NCHUNKS = ROWS_PER_W // CHUNK

_mesh = plsc.VectorSubcoreMesh(core_axis_name="c", subcore_axis_name="s")


@functools.partial(
    pl.kernel,
    out_type=jax.ShapeDtypeStruct((B_ROWS, HIDDEN_DIM), jnp.float32),
    mesh=_mesh,
    compiler_params=pltpu.CompilerParams(needs_layout_passes=False,
                                         use_tc_tiling_on_sc=False),
    scratch_types=[
        pltpu.VMEM((GROUPS, 128), jnp.int32),
        pltpu.VMEM((CHUNK, HIDDEN_DIM), jnp.float32),
        pltpu.SemaphoreType.DMA,
    ],
)
def _hubs_sc(x_hbm, table_hbm, out_hbm, idx_v, rows_v, sem):
    wid = lax.axis_index("s") * NC + lax.axis_index("c")
    base = wid * ROWS_PER_W

    @pl.loop(0, NCHUNKS)
    def _chunk(c):
        row0 = base + c * CHUNK
        g0 = pl.multiple_of(row0 // 128, 8)
        pltpu.sync_copy(x_hbm.at[pl.ds(g0, GROUPS)], idx_v)
        copies = [
            pltpu.async_copy(
                table_hbm.at[idx_v.at[g]],
                rows_v.at[pl.ds(g * 128, 128)],
                sem,
            )
            for g in range(GROUPS)
        ]
        for cp in copies:
            cp.wait()

        @pl.loop(0, CHUNK // LANES)
        def _blk(i):
            row_idx = i * LANES + lax.iota(jnp.int32, LANES)
            vs = []
            ss = jnp.zeros((LANES,), jnp.float32)
            for d in range(HIDDEN_DIM):
                col = jnp.full((LANES,), d, jnp.int32)
                vd = plsc.load_gather(rows_v, [row_idx, col])
                vs.append(vd)
                ss = ss + vd * vd
            bi = plsc.bitcast(ss, jnp.int32)
            y = plsc.bitcast(jnp.int32(0x5F3759DF) - (bi >> 1), jnp.float32)
            y = y * (1.5 - 0.5 * ss * y * y)
            y = y * (1.5 - 0.5 * ss * y * y)
            y = y * (1.5 - 0.5 * ss * y * y)
            norm = ss * y
            scale = jnp.where(norm > MAX_NORM, MAX_NORM / (norm + 1e-7),
                              jnp.float32(1.0))
            for d in range(HIDDEN_DIM):
                col = jnp.full((LANES,), d, jnp.int32)
                plsc.store_scatter(rows_v, [row_idx, col], vs[d] * scale)

        pltpu.sync_copy(rows_v, out_hbm.at[pl.ds(row0, CHUNK)])


def kernel(x, table):
    x2d = x.reshape(-1).astype(jnp.int32).reshape(B_ROWS // 128, 128)
    out = _hubs_sc(x2d, table)
    return out.reshape(x.shape[0], x.shape[1], HIDDEN_DIM)


# speedup vs baseline: 1.1712x; 1.0366x over previous
import functools

import jax
import jax.numpy as jnp
from jax import lax
from jax.experimental import pallas as pl
from jax.experimental.pallas import tpu as pltpu
from jax.experimental.pallas import tpu_sc as plsc

HIDDEN_DIM = 32
MAX_NORM = 1.0

B_ROWS = 16384 * 26
NC, NS, LANES = 2, 16, 16
NW = NC * NS
ROWS_PER_W = B_ROWS // NW
CHUNK = 256
GROUPS = CHUNK // 128
NCHUNKS = ROWS_PER_W // CHUNK
IDX_ROWS = ROWS_PER_W // 128

_mesh = plsc.VectorSubcoreMesh(core_axis_name="c", subcore_axis_name="s")


@functools.partial(
    pl.kernel,
    out_type=jax.ShapeDtypeStruct((B_ROWS, HIDDEN_DIM), jnp.float32),
    mesh=_mesh,
    compiler_params=pltpu.CompilerParams(needs_layout_passes=False,
                                         use_tc_tiling_on_sc=False),
    scratch_types=[
        pltpu.VMEM((IDX_ROWS, 128), jnp.int32),
        pltpu.VMEM((IDX_ROWS, 128), jnp.int32),
        pltpu.VMEM((CHUNK, 128), jnp.float32),
        pltpu.VMEM((CHUNK, 128), jnp.float32),
        pltpu.VMEM((CHUNK, HIDDEN_DIM), jnp.float32),
        pltpu.VMEM((CHUNK, HIDDEN_DIM), jnp.float32),
        pltpu.SemaphoreType.DMA,
        pltpu.SemaphoreType.DMA,
        pltpu.SemaphoreType.DMA,
        pltpu.SemaphoreType.DMA,
    ],
)
def _hubs_sc(x_hbm, tbl4_hbm, out_hbm, idx_all, blk_all,
             blocks_a, blocks_b, outv_a, outv_b,
             sem_ga, sem_gb, sem_oa, sem_ob):
    wid = lax.axis_index("s") * NC + lax.axis_index("c")
    base = wid * ROWS_PER_W

    pltpu.sync_copy(x_hbm.at[pl.ds(pl.multiple_of(wid * IDX_ROWS, 8),
                                   IDX_ROWS)], idx_all)

    @pl.loop(0, IDX_ROWS)
    def _mk_blk(r):
        for k in range(8):
            blk_all[r, pl.ds(k * LANES, LANES)] = (
                idx_all[r, pl.ds(k * LANES, LANES)] >> 2)

    bufs = ((blocks_a, outv_a, sem_ga, sem_oa),
            (blocks_b, outv_b, sem_gb, sem_ob))

    def fire_gathers(c, blocks, sem):
        for j in range(GROUPS):
            pltpu.async_copy(tbl4_hbm.at[blk_all.at[c * GROUPS + j]],
                             blocks.at[pl.ds(j * 128, 128)], sem)

    def wait_gathers(c, blocks, sem):
        for j in range(GROUPS):
            pltpu.make_async_copy(tbl4_hbm.at[blk_all.at[c * GROUPS + j]],
                                  blocks.at[pl.ds(j * 128, 128)], sem).wait()

    def out_slice(c):
        return out_hbm.at[pl.ds(base + c * CHUNK, CHUNK)]

    fire_gathers(0, blocks_a, sem_ga)
    fire_gathers(1, blocks_b, sem_gb)

    @pl.loop(0, NCHUNKS // 2)
    def _super(s):
        for b, (blocks, outv, sem_g, sem_o) in enumerate(bufs):
            c = s * 2 + b

            @pl.when(s >= 1)
            def _():
                pltpu.make_async_copy(outv, out_slice(c - 2), sem_o).wait()

            wait_gathers(c, blocks, sem_g)

            @pl.loop(0, CHUNK // LANES)
            def _blk(i):
                p = c * CHUNK + i * LANES
                iv = idx_all[p // 128, pl.ds(p % 128, LANES)]
                row_idx = i * LANES + lax.iota(jnp.int32, LANES)
                colv = (iv & 3) * HIDDEN_DIM
                vs = []
                ss = jnp.zeros((LANES,), jnp.float32)
                for d in range(HIDDEN_DIM):
                    vd = plsc.load_gather(blocks, [row_idx, colv + d])
                    vs.append(vd)
                    ss = ss + vd * vd
                bi = plsc.bitcast(ss, jnp.int32)
                y = plsc.bitcast(jnp.int32(0x5F3759DF) - (bi >> 1),
                                 jnp.float32)
                y = y * (1.5 - 0.5 * ss * y * y)
                y = y * (1.5 - 0.5 * ss * y * y)
                y = y * (1.5 - 0.5 * ss * y * y)
                norm = ss * y
                scale = jnp.where(norm > MAX_NORM,
                                  MAX_NORM / (norm + 1e-7), jnp.float32(1.0))
                for d in range(HIDDEN_DIM):
                    col = jnp.full((LANES,), d, jnp.int32)
                    plsc.store_scatter(outv, [row_idx, col], vs[d] * scale)

            pltpu.async_copy(outv, out_slice(c), sem_o)

            @pl.when(s < NCHUNKS // 2 - 1)
            def _():
                fire_gathers(c + 2, blocks, sem_g)

    pltpu.make_async_copy(outv_a, out_slice(NCHUNKS - 2), sem_oa).wait()
    pltpu.make_async_copy(outv_b, out_slice(NCHUNKS - 1), sem_ob).wait()


def kernel(x, table):
    x2d = x.reshape(-1).astype(jnp.int32).reshape(B_ROWS // 128, 128)
    tbl4 = table.reshape(250000, 128)
    out = _hubs_sc(x2d, tbl4)
    return out.reshape(x.shape[0], x.shape[1], HIDDEN_DIM)
